# baseline (speedup 1.0000x reference)
import functools

import jax
import jax.numpy as jnp
from jax import lax
from jax.experimental import pallas as pl
from jax.experimental.pallas import tpu as pltpu
from jax.experimental.pallas import tpu_sc as plsc

T = 8192
D = 2048
E = 16

NC = 2
NS = 16
NW = NC * NS
TW = T // NW
RK = 32


def _dispatch_body(th_hbm, eidx_hbm, sidx_hbm, off_hbm, out_hbm,
                   eidx_v, sidx_v, off_v, rows_v, sem):
    wid = lax.axis_index("s") * NC + lax.axis_index("c")
    base = wid * TW

    pltpu.sync_copy(eidx_hbm.at[pl.ds(base, TW)], eidx_v)
    pltpu.sync_copy(sidx_hbm.at[pl.ds(base, TW)], sidx_v)
    pltpu.sync_copy(off_hbm.at[pl.ds(0, E)], off_v)
    offs = off_v[...]

    for g in range(TW // 16):
        e = eidx_v[pl.ds(g * 16, 16)]
        s = sidx_v[pl.ds(g * 16, 16)]
        rows_v[pl.ds(g * 16, 16)] = offs.at[e].get(
            mode="promise_in_bounds") + s

    def row_copy(i):
        r = rows_v[pl.ds(i, 16)][0]
        return pltpu.make_async_copy(
            th_hbm.at[pl.ds(base + i, 1), :],
            out_hbm.at[pl.ds(r, 1), :], sem)

    def round_body(rnd, _):
        def fire(i, _):
            row_copy(rnd * RK + i).start()
            return ()
        lax.fori_loop(0, RK, fire, ())
        def drain(i, _):
            row_copy(rnd * RK + i).wait()
            return ()
        lax.fori_loop(0, RK, drain, ())
        return ()

    lax.fori_loop(0, TW // RK, round_body, ())


@jax.jit
def _dispatch(token_hidden, expert_idx, slot_idx, expert_offsets):
    mesh = plsc.VectorSubcoreMesh(core_axis_name="c", subcore_axis_name="s",
                                  num_cores=NC, num_subcores=NS)
    f = pl.kernel(
        _dispatch_body,
        out_type=jax.ShapeDtypeStruct((T, D), jnp.float32),
        mesh=mesh,
        scratch_types=[
            pltpu.VMEM((TW,), jnp.int32),
            pltpu.VMEM((TW,), jnp.int32),
            pltpu.VMEM((E,), jnp.int32),
            pltpu.VMEM((TW + 16,), jnp.int32),
            pltpu.SemaphoreType.DMA,
        ],
    )
    return f(token_hidden, expert_idx, slot_idx, expert_offsets)


def kernel(token_hidden, expert_idx, slot_idx, expert_offsets):
    return _dispatch(token_hidden,
                     expert_idx.astype(jnp.int32),
                     slot_idx.astype(jnp.int32),
                     expert_offsets.astype(jnp.int32))


# speedup vs baseline: 30.7643x; 30.7643x over previous
import functools

import jax
import jax.numpy as jnp
from jax import lax
from jax.experimental import pallas as pl
from jax.experimental.pallas import tpu as pltpu
from jax.experimental.pallas import tpu_sc as plsc

T = 8192
D = 2048
E = 16

NC = 2
NS = 16
NW = NC * NS
TW = T // NW
C = 16
NCHUNK = TW // C
NBUF = 3
DELAY = 0


def _dispatch_body(th_hbm, eidx_hbm, sidx_hbm, off_hbm, out_hbm,
                   eidx_v, sidx_v, off_v, buf_v, *sems):
    in_sems = sems[:NBUF]
    out_sems = sems[NBUF:]
    wid = lax.axis_index("s") * NC + lax.axis_index("c")
    base = wid * TW

    def start_in(j, slot):
        return pltpu.async_copy(
            th_hbm.at[pl.ds(base + j * C, C), :], buf_v.at[slot],
            in_sems[slot])

    in_handles = [None] * NBUF
    out_handles = [None] * NBUF
    for j in range(min(NBUF, NCHUNK)):
        in_handles[j] = start_in(j, j)

    pltpu.sync_copy(eidx_hbm.at[pl.ds(base, TW)], eidx_v)
    pltpu.sync_copy(sidx_hbm.at[pl.ds(base, TW)], sidx_v)
    pltpu.sync_copy(off_hbm.at[pl.ds(0, E)], off_v)
    offs = off_v[...]

    for j in range(NCHUNK):
        slot = j % NBUF
        in_handles[slot].wait()
        e = eidx_v[pl.ds(j * C, C)]
        s = sidx_v[pl.ds(j * C, C)]
        rows = offs.at[e].get(mode="promise_in_bounds") + s
        out_handles[slot] = pltpu.async_copy(
            buf_v.at[slot], out_hbm.at[rows], out_sems[slot])
        pj = j - DELAY
        nj = pj + NBUF
        if pj >= 0 and nj < NCHUNK:
            out_handles[pj % NBUF].wait()
            in_handles[nj % NBUF] = start_in(nj, nj % NBUF)

    for j in range(max(0, NCHUNK - NBUF), NCHUNK):
        out_handles[j % NBUF].wait()


@jax.jit
def _dispatch(token_hidden, expert_idx, slot_idx, expert_offsets):
    mesh = plsc.VectorSubcoreMesh(core_axis_name="c", subcore_axis_name="s",
                                  num_cores=NC, num_subcores=NS)
    f = pl.kernel(
        _dispatch_body,
        out_type=jax.ShapeDtypeStruct((T, D), jnp.float32),
        mesh=mesh,
        scratch_types=[
            pltpu.VMEM((TW,), jnp.int32),
            pltpu.VMEM((TW,), jnp.int32),
            pltpu.VMEM((E,), jnp.int32),
            pltpu.VMEM((NBUF, C, D), jnp.float32),
            *([pltpu.SemaphoreType.DMA] * (2 * NBUF)),
        ],
    )
    return f(token_hidden, expert_idx, slot_idx, expert_offsets)


def kernel(token_hidden, expert_idx, slot_idx, expert_offsets):
    return _dispatch(token_hidden,
                     expert_idx.astype(jnp.int32),
                     slot_idx.astype(jnp.int32),
                     expert_offsets.astype(jnp.int32))
